# baseline (speedup 1.0000x reference)
import jax
import jax.numpy as jnp
from jax import lax
from jax.experimental import pallas as pl
from jax.experimental.pallas import tpu as pltpu
from jax.experimental.pallas import tpu_sc as plsc

NNODE = 10000
D = 128
NC, NS = 2, 16
NW = NC * NS
CH = 128
CPW = 80
EPW = CPW * CH
EPAD = NW * EPW
NPAD = 10240
RPW = NPAD // NW
RPT = NPAD // NS
BLK = 512
NCOMBO = 16


def _mesh():
    return plsc.VectorSubcoreMesh(
        core_axis_name="c", subcore_axis_name="s", num_cores=NC, num_subcores=NS
    )


def _pre_body(emb_hbm, xi2d, dst2d, ea02d, ea12d,
              h0_hbm, cnt_hbm,
              idx64, rows64, dst_sl, ea0_sl, ea1_sl, cidx, ones_v, zbuf1d,
              cnt_sp, sem):
    c = lax.axis_index("c")
    s = lax.axis_index("s")
    w = s * NC + c

    for k in range(8):
        ones_v[pl.ds(k * 16, 16)] = jnp.ones((16,), jnp.float32)

    @pl.loop(0, 640)
    def _zero(i):
        zbuf1d[pl.ds(i * 16, 16)] = jnp.zeros((16,), jnp.float32)

    pltpu.sync_copy(zbuf1d, cnt_sp.at[pl.ds(s * 10240, 10240)])

    @pl.loop(0, 5)
    def _h0(t):
        pltpu.sync_copy(xi2d.at[w * 5 + t], idx64)
        pltpu.async_copy(emb_hbm.at[idx64], rows64, sem).wait()
        pltpu.sync_copy(rows64, h0_hbm.at[pl.ds(w * RPW + t * 64, 64)])

    plsc.subcore_barrier()

    pltpu.sync_copy(dst2d.at[pl.ds(w * CPW, CPW)], dst_sl)
    pltpu.sync_copy(ea02d.at[pl.ds(w * CPW, CPW)], ea0_sl)
    pltpu.sync_copy(ea12d.at[pl.ds(w * CPW, CPW)], ea1_sl)

    @pl.loop(0, CPW)
    def _cnt(j):
        for k in range(8):
            sl = pl.ds(k * 16, 16)
            cidx[0, sl] = dst_sl[j, sl] * 16 + ea0_sl[j, sl] * 3 + ea1_sl[j, sl]
        pltpu.sync_copy(ones_v, cnt_sp.at[cidx.at[0]], add=True)

    plsc.subcore_barrier()
    pltpu.sync_copy(cnt_sp.at[pl.ds(s * 10240, 10240)],
                    cnt_hbm.at[c, pl.ds(s * 10240, 10240)])


def _pre(emb, xi2d, dst2d, ea02d, ea12d):
    return pl.kernel(
        _pre_body,
        out_type=(
            jax.ShapeDtypeStruct((NPAD, D), jnp.float32),
            jax.ShapeDtypeStruct((NC, NPAD * NCOMBO), jnp.float32),
        ),
        mesh=_mesh(),
        scratch_types=[
            pltpu.VMEM((64,), jnp.int32),
            pltpu.VMEM((64, D), jnp.float32),
            pltpu.VMEM((CPW, CH), jnp.int32),
            pltpu.VMEM((CPW, CH), jnp.int32),
            pltpu.VMEM((CPW, CH), jnp.int32),
            pltpu.VMEM((1, CH), jnp.int32),
            pltpu.VMEM((CH,), jnp.float32),
            pltpu.VMEM((10240,), jnp.float32),
            pltpu.VMEM_SHARED((NPAD * NCOMBO,), jnp.float32),
            pltpu.SemaphoreType.DMA,
        ],
    )(emb, xi2d, dst2d, ea02d, ea12d)


CPW0 = 48
CPW1 = 2 * CPW - CPW0


def _agg_body(h_hbm, src2d, dst2d, out_hbm,
              src_sl, dst_sl, rows, zbuf, agg_sp, sem):
    c = lax.axis_index("c")
    s = lax.axis_index("s")

    @pl.loop(0, 8)
    def _zb(i):
        for k in range(8):
            zbuf[i, pl.ds(k * 16, 16)] = jnp.zeros((16,), jnp.float32)

    @pl.loop(0, 80)
    def _za(i):
        pltpu.sync_copy(zbuf, agg_sp.at[pl.ds(s * RPT + i * 8, 8)])

    plsc.subcore_barrier()

    base = s * (2 * CPW) + c * CPW0
    cpw = CPW0 + c * (CPW1 - CPW0)
    pltpu.sync_copy(src2d.at[pl.ds(base, CPW1)], src_sl)
    pltpu.sync_copy(dst2d.at[pl.ds(base, CPW1)], dst_sl)

    @pl.loop(0, cpw)
    def _edge(j):
        pltpu.async_copy(h_hbm.at[src_sl.at[j]], rows, sem).wait()
        pltpu.sync_copy(rows, agg_sp.at[dst_sl.at[j]], add=True)

    plsc.subcore_barrier()
    pltpu.sync_copy(agg_sp.at[pl.ds(s * RPT, RPT)],
                    out_hbm.at[c, pl.ds(s * RPT, RPT)])


def _edge_agg(h_pad, src2d, dst2d):
    return pl.kernel(
        _agg_body,
        out_type=jax.ShapeDtypeStruct((NC, NPAD, D), jnp.float32),
        mesh=_mesh(),
        scratch_types=[
            pltpu.VMEM((CPW1, CH), jnp.int32),
            pltpu.VMEM((CPW1, CH), jnp.int32),
            pltpu.VMEM((CH, D), jnp.float32),
            pltpu.VMEM((8, D), jnp.float32),
            pltpu.VMEM_SHARED((NPAD, D), jnp.float32),
            pltpu.SemaphoreType.DMA,
        ],
    )(h_pad, src2d, dst2d)


def _mlp_body(p_ref, h_ref, cnt_ref, t_ref, w1_ref, b1_ref, w2_ref, b2_ref,
              h2_ref, stats_ref):
    i = pl.program_id(0)
    hp = jax.lax.Precision.HIGHEST
    bf = jnp.bfloat16
    f32 = jnp.float32
    cnt = cnt_ref[0] + cnt_ref[1]
    ee = jnp.dot(cnt, t_ref[...], preferred_element_type=f32, precision=hp)
    agg = p_ref[0] + p_ref[1] + h_ref[...] + ee + t_ref[12:13, :]
    hmid = jnp.maximum(
        jnp.dot(agg.astype(bf), w1_ref[...].astype(bf),
                preferred_element_type=f32) + b1_ref[...], 0.0)
    h2 = jnp.dot(hmid.astype(bf), w2_ref[...].astype(bf),
                 preferred_element_type=f32) + b2_ref[...]
    h2_ref[...] = h2
    rows = i * BLK + lax.broadcasted_iota(jnp.int32, (BLK, 1), 0)
    m = (rows < NNODE).astype(jnp.float32)
    hm = h2 * m

    @pl.when(i == 0)
    def _():
        stats_ref[...] = jnp.zeros_like(stats_ref)

    stats_ref[0:1, :] += jnp.sum(hm, axis=0, keepdims=True)
    stats_ref[1:2, :] += jnp.sum(hm * h2, axis=0, keepdims=True)


def _mlp(p, h, cnt3, t, w1, b1, w2, b2):
    grid = NPAD // BLK
    return pl.pallas_call(
        _mlp_body,
        grid=(grid,),
        in_specs=[
            pl.BlockSpec((NC, BLK, D), lambda i: (0, i, 0)),
            pl.BlockSpec((BLK, D), lambda i: (i, 0)),
            pl.BlockSpec((NC, BLK, NCOMBO), lambda i: (0, i, 0)),
            pl.BlockSpec((NCOMBO, D), lambda i: (0, 0)),
            pl.BlockSpec((D, 2 * D), lambda i: (0, 0)),
            pl.BlockSpec((1, 2 * D), lambda i: (0, 0)),
            pl.BlockSpec((2 * D, D), lambda i: (0, 0)),
            pl.BlockSpec((1, D), lambda i: (0, 0)),
        ],
        out_specs=[
            pl.BlockSpec((BLK, D), lambda i: (i, 0)),
            pl.BlockSpec((8, D), lambda i: (0, 0)),
        ],
        out_shape=[
            jax.ShapeDtypeStruct((NPAD, D), jnp.float32),
            jax.ShapeDtypeStruct((8, D), jnp.float32),
        ],
    )(p, h, cnt3, t, w1, b1, w2, b2)


def _bn_body(relu, h2_ref, stats_ref, g_ref, be_ref, out_ref):
    inv_n = 1.0 / NNODE
    mu = stats_ref[0:1, :] * inv_n
    ex2 = stats_ref[1:2, :] * inv_n
    var = ex2 - mu * mu
    rstd = lax.rsqrt(var + 1e-5)
    y = (h2_ref[...] - mu) * (g_ref[...] * rstd) + be_ref[...]
    if relu:
        y = jnp.maximum(y, 0.0)
    out_ref[...] = y


def _bn(relu, h2, stats, g, be):
    import functools
    return pl.pallas_call(
        functools.partial(_bn_body, relu),
        grid=(NPAD // BLK,),
        in_specs=[
            pl.BlockSpec((BLK, D), lambda i: (i, 0)),
            pl.BlockSpec((8, D), lambda i: (0, 0)),
            pl.BlockSpec((1, D), lambda i: (0, 0)),
            pl.BlockSpec((1, D), lambda i: (0, 0)),
        ],
        out_specs=pl.BlockSpec((BLK, D), lambda i: (i, 0)),
        out_shape=jax.ShapeDtypeStruct((NPAD, D), jnp.float32),
    )(h2, stats, g, be)


def kernel(x, edge_index, edge_attr, emb, e1_0, e2_0, W1_0, b1_0, W2_0, b2_0,
           g_0, be_0, e1_1, e2_1, W1_1, b1_1, W2_1, b2_1, g_1, be_1):
    i32 = jnp.int32
    xi = jnp.zeros((NPAD,), i32).at[:NNODE].set(x[:, 0].astype(i32))
    src = edge_index[0].astype(i32)
    dst = edge_index[1].astype(i32)
    e = src.shape[0]
    pad = EPAD - e
    src_p = jnp.concatenate([src, jnp.zeros((pad,), i32)])
    dst_p = jnp.concatenate([dst, jnp.full((pad,), NPAD - 1, i32)])
    ea0_p = jnp.concatenate([edge_attr[:, 0].astype(i32), jnp.full((pad,), 5, i32)])
    ea1_p = jnp.concatenate([edge_attr[:, 1].astype(i32), jnp.zeros((pad,), i32)])
    src2d = src_p.reshape(NW * CPW, CH)
    dst2d = dst_p.reshape(NW * CPW, CH)
    ea02d = ea0_p.reshape(NW * CPW, CH)
    ea12d = ea1_p.reshape(NW * CPW, CH)
    xi2d = xi.reshape(NPAD // 64, 64)

    r5 = jnp.repeat(jnp.arange(5), 3)
    r3 = jnp.tile(jnp.arange(3), 5)
    t0 = jnp.concatenate([e1_0[r5] + e2_0[r3], jnp.zeros((1, D), jnp.float32)], 0)
    t1 = jnp.concatenate([e1_1[r5] + e2_1[r3], jnp.zeros((1, D), jnp.float32)], 0)

    h0, cnt_flat = _pre(emb, xi2d, dst2d, ea02d, ea12d)
    cnt3 = cnt_flat.reshape(NC, NPAD, NCOMBO)

    p0 = _edge_agg(h0, src2d, dst2d)
    h2a, stats0 = _mlp(p0, h0, cnt3, t0, W1_0, b1_0.reshape(1, -1), W2_0,
                       b2_0.reshape(1, -1))
    h1 = _bn(True, h2a, stats0, g_0.reshape(1, -1), be_0.reshape(1, -1))

    p1 = _edge_agg(h1, src2d, dst2d)
    h2b, stats1 = _mlp(p1, h1, cnt3, t1, W1_1, b1_1.reshape(1, -1), W2_1,
                       b2_1.reshape(1, -1))
    out = _bn(False, h2b, stats1, g_1.reshape(1, -1), be_1.reshape(1, -1))
    return out[:NNODE]


# speedup vs baseline: 1.1833x; 1.1833x over previous
import jax
import jax.numpy as jnp
from jax import lax
from jax.experimental import pallas as pl
from jax.experimental.pallas import tpu as pltpu
from jax.experimental.pallas import tpu_sc as plsc

NNODE = 10000
D = 128
NC, NS = 2, 16
NW = NC * NS
CH = 128
CPW = 80
EPW = CPW * CH
EPAD = NW * EPW
NPAD = 10240
RPW = NPAD // NW
RPT = NPAD // NS
BLK = 512
NCOMBO = 16


def _mesh():
    return plsc.VectorSubcoreMesh(
        core_axis_name="c", subcore_axis_name="s", num_cores=NC, num_subcores=NS
    )


def _pre_body(emb_hbm, xi2d, dst2d, ea02d, ea12d,
              h0_hbm, cnt_hbm,
              idx64, rows64, dst_sl, ea0_sl, ea1_sl, cidx, ones_v, zbuf1d,
              cnt_sp, sem):
    c = lax.axis_index("c")
    s = lax.axis_index("s")
    w = s * NC + c

    for k in range(8):
        ones_v[pl.ds(k * 16, 16)] = jnp.ones((16,), jnp.float32)

    @pl.loop(0, 640)
    def _zero(i):
        zbuf1d[pl.ds(i * 16, 16)] = jnp.zeros((16,), jnp.float32)

    pltpu.sync_copy(zbuf1d, cnt_sp.at[pl.ds(s * 10240, 10240)])

    @pl.loop(0, 5)
    def _h0(t):
        pltpu.sync_copy(xi2d.at[w * 5 + t], idx64)
        pltpu.async_copy(emb_hbm.at[idx64], rows64, sem).wait()
        pltpu.sync_copy(rows64, h0_hbm.at[pl.ds(w * RPW + t * 64, 64)])

    plsc.subcore_barrier()

    pltpu.sync_copy(dst2d.at[pl.ds(w * CPW, CPW)], dst_sl)
    pltpu.sync_copy(ea02d.at[pl.ds(w * CPW, CPW)], ea0_sl)
    pltpu.sync_copy(ea12d.at[pl.ds(w * CPW, CPW)], ea1_sl)

    @pl.loop(0, CPW)
    def _cnt(j):
        for k in range(8):
            sl = pl.ds(k * 16, 16)
            cidx[0, sl] = dst_sl[j, sl] * 16 + ea0_sl[j, sl] * 3 + ea1_sl[j, sl]
        pltpu.sync_copy(ones_v, cnt_sp.at[cidx.at[0]], add=True)

    plsc.subcore_barrier()
    pltpu.sync_copy(cnt_sp.at[pl.ds(s * 10240, 10240)],
                    cnt_hbm.at[c, pl.ds(s * 10240, 10240)])


def _pre(emb, xi2d, dst2d, ea02d, ea12d):
    return pl.kernel(
        _pre_body,
        out_type=(
            jax.ShapeDtypeStruct((NPAD, D), jnp.float32),
            jax.ShapeDtypeStruct((NC, NPAD * NCOMBO), jnp.float32),
        ),
        mesh=_mesh(),
        scratch_types=[
            pltpu.VMEM((64,), jnp.int32),
            pltpu.VMEM((64, D), jnp.float32),
            pltpu.VMEM((CPW, CH), jnp.int32),
            pltpu.VMEM((CPW, CH), jnp.int32),
            pltpu.VMEM((CPW, CH), jnp.int32),
            pltpu.VMEM((1, CH), jnp.int32),
            pltpu.VMEM((CH,), jnp.float32),
            pltpu.VMEM((10240,), jnp.float32),
            pltpu.VMEM_SHARED((NPAD * NCOMBO,), jnp.float32),
            pltpu.SemaphoreType.DMA,
        ],
    )(emb, xi2d, dst2d, ea02d, ea12d)


CPW0 = 112
CPW1 = 2 * CPW - CPW0
CPWMX = max(CPW0, CPW1)


def _agg_body(h_hbm, src2d, dst2d, out_hbm,
              src_sl, dst_sl, rows, zbuf, agg_sp, sem):
    c = lax.axis_index("c")
    s = lax.axis_index("s")

    @pl.loop(0, 8)
    def _zb(i):
        for k in range(8):
            zbuf[i, pl.ds(k * 16, 16)] = jnp.zeros((16,), jnp.float32)

    @pl.loop(0, 80)
    def _za(i):
        pltpu.sync_copy(zbuf, agg_sp.at[pl.ds(s * RPT + i * 8, 8)])

    plsc.subcore_barrier()

    base = s * (2 * CPW) + c * CPW0
    cpw = CPW0 + c * (CPW1 - CPW0)
    pltpu.sync_copy(src2d.at[pl.ds(base, CPWMX)], src_sl)
    pltpu.sync_copy(dst2d.at[pl.ds(base, CPWMX)], dst_sl)

    @pl.loop(0, cpw)
    def _edge(j):
        pltpu.async_copy(h_hbm.at[src_sl.at[j]], rows, sem).wait()
        pltpu.sync_copy(rows, agg_sp.at[dst_sl.at[j]], add=True)

    plsc.subcore_barrier()
    pltpu.sync_copy(agg_sp.at[pl.ds(s * RPT, RPT)],
                    out_hbm.at[c, pl.ds(s * RPT, RPT)])


def _edge_agg(h_pad, src2d, dst2d):
    return pl.kernel(
        _agg_body,
        out_type=jax.ShapeDtypeStruct((NC, NPAD, D), jnp.float32),
        mesh=_mesh(),
        scratch_types=[
            pltpu.VMEM((CPWMX, CH), jnp.int32),
            pltpu.VMEM((CPWMX, CH), jnp.int32),
            pltpu.VMEM((CH, D), jnp.float32),
            pltpu.VMEM((8, D), jnp.float32),
            pltpu.VMEM_SHARED((NPAD, D), jnp.float32),
            pltpu.SemaphoreType.DMA,
        ],
    )(h_pad, src2d, dst2d)


def _mlp_body(p_ref, h_ref, cnt_ref, t_ref, w1_ref, b1_ref, w2_ref, b2_ref,
              h2_ref, stats_ref):
    i = pl.program_id(0)
    hp = jax.lax.Precision.HIGHEST
    bf = jnp.bfloat16
    f32 = jnp.float32
    cnt = cnt_ref[0] + cnt_ref[1]
    ee = jnp.dot(cnt, t_ref[...], preferred_element_type=f32, precision=hp)
    agg = p_ref[0] + p_ref[1] + h_ref[...] + ee + t_ref[12:13, :]
    hmid = jnp.maximum(
        jnp.dot(agg.astype(bf), w1_ref[...].astype(bf),
                preferred_element_type=f32) + b1_ref[...], 0.0)
    h2 = jnp.dot(hmid.astype(bf), w2_ref[...].astype(bf),
                 preferred_element_type=f32) + b2_ref[...]
    h2_ref[...] = h2
    rows = i * BLK + lax.broadcasted_iota(jnp.int32, (BLK, 1), 0)
    m = (rows < NNODE).astype(jnp.float32)
    hm = h2 * m

    @pl.when(i == 0)
    def _():
        stats_ref[...] = jnp.zeros_like(stats_ref)

    stats_ref[0:1, :] += jnp.sum(hm, axis=0, keepdims=True)
    stats_ref[1:2, :] += jnp.sum(hm * h2, axis=0, keepdims=True)


def _mlp(p, h, cnt3, t, w1, b1, w2, b2):
    grid = NPAD // BLK
    return pl.pallas_call(
        _mlp_body,
        grid=(grid,),
        in_specs=[
            pl.BlockSpec((NC, BLK, D), lambda i: (0, i, 0)),
            pl.BlockSpec((BLK, D), lambda i: (i, 0)),
            pl.BlockSpec((NC, BLK, NCOMBO), lambda i: (0, i, 0)),
            pl.BlockSpec((NCOMBO, D), lambda i: (0, 0)),
            pl.BlockSpec((D, 2 * D), lambda i: (0, 0)),
            pl.BlockSpec((1, 2 * D), lambda i: (0, 0)),
            pl.BlockSpec((2 * D, D), lambda i: (0, 0)),
            pl.BlockSpec((1, D), lambda i: (0, 0)),
        ],
        out_specs=[
            pl.BlockSpec((BLK, D), lambda i: (i, 0)),
            pl.BlockSpec((8, D), lambda i: (0, 0)),
        ],
        out_shape=[
            jax.ShapeDtypeStruct((NPAD, D), jnp.float32),
            jax.ShapeDtypeStruct((8, D), jnp.float32),
        ],
    )(p, h, cnt3, t, w1, b1, w2, b2)


def _bn_body(relu, h2_ref, stats_ref, g_ref, be_ref, out_ref):
    inv_n = 1.0 / NNODE
    mu = stats_ref[0:1, :] * inv_n
    ex2 = stats_ref[1:2, :] * inv_n
    var = ex2 - mu * mu
    rstd = lax.rsqrt(var + 1e-5)
    y = (h2_ref[...] - mu) * (g_ref[...] * rstd) + be_ref[...]
    if relu:
        y = jnp.maximum(y, 0.0)
    out_ref[...] = y


def _bn(relu, h2, stats, g, be):
    import functools
    return pl.pallas_call(
        functools.partial(_bn_body, relu),
        grid=(NPAD // BLK,),
        in_specs=[
            pl.BlockSpec((BLK, D), lambda i: (i, 0)),
            pl.BlockSpec((8, D), lambda i: (0, 0)),
            pl.BlockSpec((1, D), lambda i: (0, 0)),
            pl.BlockSpec((1, D), lambda i: (0, 0)),
        ],
        out_specs=pl.BlockSpec((BLK, D), lambda i: (i, 0)),
        out_shape=jax.ShapeDtypeStruct((NPAD, D), jnp.float32),
    )(h2, stats, g, be)


def kernel(x, edge_index, edge_attr, emb, e1_0, e2_0, W1_0, b1_0, W2_0, b2_0,
           g_0, be_0, e1_1, e2_1, W1_1, b1_1, W2_1, b2_1, g_1, be_1):
    i32 = jnp.int32
    xi = jnp.zeros((NPAD,), i32).at[:NNODE].set(x[:, 0].astype(i32))
    src = edge_index[0].astype(i32)
    dst = edge_index[1].astype(i32)
    e = src.shape[0]
    pad = EPAD - e
    src_p = jnp.concatenate([src, jnp.zeros((pad,), i32)])
    dst_p = jnp.concatenate([dst, jnp.full((pad,), NPAD - 1, i32)])
    ea0_p = jnp.concatenate([edge_attr[:, 0].astype(i32), jnp.full((pad,), 5, i32)])
    ea1_p = jnp.concatenate([edge_attr[:, 1].astype(i32), jnp.zeros((pad,), i32)])
    zrows = jnp.zeros((CPWMX, CH), i32)
    src2d = jnp.concatenate([src_p.reshape(NW * CPW, CH), zrows])
    dst2d = jnp.concatenate([dst_p.reshape(NW * CPW, CH), zrows])
    ea02d = ea0_p.reshape(NW * CPW, CH)
    ea12d = ea1_p.reshape(NW * CPW, CH)
    xi2d = xi.reshape(NPAD // 64, 64)

    r5 = jnp.repeat(jnp.arange(5), 3)
    r3 = jnp.tile(jnp.arange(3), 5)
    t0 = jnp.concatenate([e1_0[r5] + e2_0[r3], jnp.zeros((1, D), jnp.float32)], 0)
    t1 = jnp.concatenate([e1_1[r5] + e2_1[r3], jnp.zeros((1, D), jnp.float32)], 0)

    h0, cnt_flat = _pre(emb, xi2d, dst2d, ea02d, ea12d)
    cnt3 = cnt_flat.reshape(NC, NPAD, NCOMBO)

    p0 = _edge_agg(h0, src2d, dst2d)
    h2a, stats0 = _mlp(p0, h0, cnt3, t0, W1_0, b1_0.reshape(1, -1), W2_0,
                       b2_0.reshape(1, -1))
    h1 = _bn(True, h2a, stats0, g_0.reshape(1, -1), be_0.reshape(1, -1))

    p1 = _edge_agg(h1, src2d, dst2d)
    h2b, stats1 = _mlp(p1, h1, cnt3, t1, W1_1, b1_1.reshape(1, -1), W2_1,
                       b2_1.reshape(1, -1))
    out = _bn(False, h2b, stats1, g_1.reshape(1, -1), be_1.reshape(1, -1))
    return out[:NNODE]


# speedup vs baseline: 1.1877x; 1.0037x over previous
import jax
import jax.numpy as jnp
from jax import lax
from jax.experimental import pallas as pl
from jax.experimental.pallas import tpu as pltpu
from jax.experimental.pallas import tpu_sc as plsc

NNODE = 10000
D = 128
NC, NS = 2, 16
NW = NC * NS
CH = 128
CPW = 80
EPW = CPW * CH
EPAD = NW * EPW
NPAD = 10240
RPW = NPAD // NW
RPT = NPAD // NS
BLK = 512
NCOMBO = 16


def _mesh():
    return plsc.VectorSubcoreMesh(
        core_axis_name="c", subcore_axis_name="s", num_cores=NC, num_subcores=NS
    )


def _pre_body(emb_hbm, xi2d, dst2d, ea02d, ea12d,
              h0_hbm, cnt_hbm,
              idx64, rows64, dst_sl, ea0_sl, ea1_sl, cidx, ones_v, zbuf1d,
              cnt_sp, sem):
    c = lax.axis_index("c")
    s = lax.axis_index("s")
    w = s * NC + c

    for k in range(8):
        ones_v[pl.ds(k * 16, 16)] = jnp.ones((16,), jnp.float32)

    @pl.loop(0, 640)
    def _zero(i):
        zbuf1d[pl.ds(i * 16, 16)] = jnp.zeros((16,), jnp.float32)

    pltpu.sync_copy(zbuf1d, cnt_sp.at[pl.ds(s * 10240, 10240)])

    @pl.loop(0, 5)
    def _h0(t):
        pltpu.sync_copy(xi2d.at[w * 5 + t], idx64)
        pltpu.async_copy(emb_hbm.at[idx64], rows64, sem).wait()
        pltpu.sync_copy(rows64, h0_hbm.at[pl.ds(w * RPW + t * 64, 64)])

    plsc.subcore_barrier()

    pltpu.sync_copy(dst2d.at[pl.ds(w * CPW, CPW)], dst_sl)
    pltpu.sync_copy(ea02d.at[pl.ds(w * CPW, CPW)], ea0_sl)
    pltpu.sync_copy(ea12d.at[pl.ds(w * CPW, CPW)], ea1_sl)

    @pl.loop(0, CPW)
    def _cnt(j):
        for k in range(8):
            sl = pl.ds(k * 16, 16)
            cidx[0, sl] = dst_sl[j, sl] * 16 + ea0_sl[j, sl] * 3 + ea1_sl[j, sl]
        pltpu.sync_copy(ones_v, cnt_sp.at[cidx.at[0]], add=True)

    plsc.subcore_barrier()
    pltpu.sync_copy(cnt_sp.at[pl.ds(s * 10240, 10240)],
                    cnt_hbm.at[c, pl.ds(s * 10240, 10240)])


def _pre(emb, xi2d, dst2d, ea02d, ea12d):
    return pl.kernel(
        _pre_body,
        out_type=(
            jax.ShapeDtypeStruct((NPAD, D), jnp.float32),
            jax.ShapeDtypeStruct((NC, NPAD * NCOMBO), jnp.float32),
        ),
        mesh=_mesh(),
        scratch_types=[
            pltpu.VMEM((64,), jnp.int32),
            pltpu.VMEM((64, D), jnp.float32),
            pltpu.VMEM((CPW, CH), jnp.int32),
            pltpu.VMEM((CPW, CH), jnp.int32),
            pltpu.VMEM((CPW, CH), jnp.int32),
            pltpu.VMEM((1, CH), jnp.int32),
            pltpu.VMEM((CH,), jnp.float32),
            pltpu.VMEM((10240,), jnp.float32),
            pltpu.VMEM_SHARED((NPAD * NCOMBO,), jnp.float32),
            pltpu.SemaphoreType.DMA,
        ],
    )(emb, xi2d, dst2d, ea02d, ea12d)


CPW0 = 120
CPW1 = 2 * CPW - CPW0
CPWMX = max(CPW0, CPW1)


def _agg_body(h_hbm, src2d, dst2d, out_hbm,
              src_sl, dst_sl, rows, zbuf, agg_sp, sem):
    c = lax.axis_index("c")
    s = lax.axis_index("s")

    @pl.loop(0, 8)
    def _zb(i):
        for k in range(8):
            zbuf[i, pl.ds(k * 16, 16)] = jnp.zeros((16,), jnp.float32)

    @pl.loop(0, 80)
    def _za(i):
        pltpu.sync_copy(zbuf, agg_sp.at[pl.ds(s * RPT + i * 8, 8)])

    plsc.subcore_barrier()

    base = s * (2 * CPW) + c * CPW0
    cpw = CPW0 + c * (CPW1 - CPW0)
    pltpu.sync_copy(src2d.at[pl.ds(base, CPWMX)], src_sl)
    pltpu.sync_copy(dst2d.at[pl.ds(base, CPWMX)], dst_sl)

    @pl.loop(0, cpw)
    def _edge(j):
        pltpu.async_copy(h_hbm.at[src_sl.at[j]], rows, sem).wait()
        pltpu.sync_copy(rows, agg_sp.at[dst_sl.at[j]], add=True)

    plsc.subcore_barrier()
    pltpu.sync_copy(agg_sp.at[pl.ds(s * RPT, RPT)],
                    out_hbm.at[c, pl.ds(s * RPT, RPT)])


def _edge_agg(h_pad, src2d, dst2d):
    return pl.kernel(
        _agg_body,
        out_type=jax.ShapeDtypeStruct((NC, NPAD, D), jnp.float32),
        mesh=_mesh(),
        scratch_types=[
            pltpu.VMEM((CPWMX, CH), jnp.int32),
            pltpu.VMEM((CPWMX, CH), jnp.int32),
            pltpu.VMEM((CH, D), jnp.float32),
            pltpu.VMEM((8, D), jnp.float32),
            pltpu.VMEM_SHARED((NPAD, D), jnp.float32),
            pltpu.SemaphoreType.DMA,
        ],
    )(h_pad, src2d, dst2d)


def _mlp_body(p_ref, h_ref, cnt_ref, t_ref, w1_ref, b1_ref, w2_ref, b2_ref,
              h2_ref, stats_ref):
    i = pl.program_id(0)
    hp = jax.lax.Precision.HIGHEST
    bf = jnp.bfloat16
    f32 = jnp.float32
    cnt = cnt_ref[0] + cnt_ref[1]
    ee = jnp.dot(cnt, t_ref[...], preferred_element_type=f32, precision=hp)
    agg = p_ref[0] + p_ref[1] + h_ref[...] + ee + t_ref[12:13, :]
    hmid = jnp.maximum(
        jnp.dot(agg.astype(bf), w1_ref[...].astype(bf),
                preferred_element_type=f32) + b1_ref[...], 0.0)
    h2 = jnp.dot(hmid.astype(bf), w2_ref[...].astype(bf),
                 preferred_element_type=f32) + b2_ref[...]
    h2_ref[...] = h2
    rows = i * BLK + lax.broadcasted_iota(jnp.int32, (BLK, 1), 0)
    m = (rows < NNODE).astype(jnp.float32)
    hm = h2 * m

    @pl.when(i == 0)
    def _():
        stats_ref[...] = jnp.zeros_like(stats_ref)

    stats_ref[0:1, :] += jnp.sum(hm, axis=0, keepdims=True)
    stats_ref[1:2, :] += jnp.sum(hm * h2, axis=0, keepdims=True)


def _mlp(p, h, cnt3, t, w1, b1, w2, b2):
    grid = NPAD // BLK
    return pl.pallas_call(
        _mlp_body,
        grid=(grid,),
        in_specs=[
            pl.BlockSpec((NC, BLK, D), lambda i: (0, i, 0)),
            pl.BlockSpec((BLK, D), lambda i: (i, 0)),
            pl.BlockSpec((NC, BLK, NCOMBO), lambda i: (0, i, 0)),
            pl.BlockSpec((NCOMBO, D), lambda i: (0, 0)),
            pl.BlockSpec((D, 2 * D), lambda i: (0, 0)),
            pl.BlockSpec((1, 2 * D), lambda i: (0, 0)),
            pl.BlockSpec((2 * D, D), lambda i: (0, 0)),
            pl.BlockSpec((1, D), lambda i: (0, 0)),
        ],
        out_specs=[
            pl.BlockSpec((BLK, D), lambda i: (i, 0)),
            pl.BlockSpec((8, D), lambda i: (0, 0)),
        ],
        out_shape=[
            jax.ShapeDtypeStruct((NPAD, D), jnp.float32),
            jax.ShapeDtypeStruct((8, D), jnp.float32),
        ],
    )(p, h, cnt3, t, w1, b1, w2, b2)


def _bn_body(relu, h2_ref, stats_ref, g_ref, be_ref, out_ref):
    inv_n = 1.0 / NNODE
    mu = stats_ref[0:1, :] * inv_n
    ex2 = stats_ref[1:2, :] * inv_n
    var = ex2 - mu * mu
    rstd = lax.rsqrt(var + 1e-5)
    y = (h2_ref[...] - mu) * (g_ref[...] * rstd) + be_ref[...]
    if relu:
        y = jnp.maximum(y, 0.0)
    out_ref[...] = y


def _bn(relu, h2, stats, g, be):
    import functools
    return pl.pallas_call(
        functools.partial(_bn_body, relu),
        grid=(NPAD // BLK,),
        in_specs=[
            pl.BlockSpec((BLK, D), lambda i: (i, 0)),
            pl.BlockSpec((8, D), lambda i: (0, 0)),
            pl.BlockSpec((1, D), lambda i: (0, 0)),
            pl.BlockSpec((1, D), lambda i: (0, 0)),
        ],
        out_specs=pl.BlockSpec((BLK, D), lambda i: (i, 0)),
        out_shape=jax.ShapeDtypeStruct((NPAD, D), jnp.float32),
    )(h2, stats, g, be)


def kernel(x, edge_index, edge_attr, emb, e1_0, e2_0, W1_0, b1_0, W2_0, b2_0,
           g_0, be_0, e1_1, e2_1, W1_1, b1_1, W2_1, b2_1, g_1, be_1):
    i32 = jnp.int32
    xi = jnp.zeros((NPAD,), i32).at[:NNODE].set(x[:, 0].astype(i32))
    src = edge_index[0].astype(i32)
    dst = edge_index[1].astype(i32)
    e = src.shape[0]
    pad = EPAD - e
    src_p = jnp.concatenate([src, jnp.zeros((pad,), i32)])
    dst_p = jnp.concatenate([dst, jnp.full((pad,), NPAD - 1, i32)])
    ea0_p = jnp.concatenate([edge_attr[:, 0].astype(i32), jnp.full((pad,), 5, i32)])
    ea1_p = jnp.concatenate([edge_attr[:, 1].astype(i32), jnp.zeros((pad,), i32)])
    zrows = jnp.zeros((CPWMX, CH), i32)
    src2d = jnp.concatenate([src_p.reshape(NW * CPW, CH), zrows])
    dst2d = jnp.concatenate([dst_p.reshape(NW * CPW, CH), zrows])
    ea02d = ea0_p.reshape(NW * CPW, CH)
    ea12d = ea1_p.reshape(NW * CPW, CH)
    xi2d = xi.reshape(NPAD // 64, 64)

    r5 = jnp.repeat(jnp.arange(5), 3)
    r3 = jnp.tile(jnp.arange(3), 5)
    t0 = jnp.concatenate([e1_0[r5] + e2_0[r3], jnp.zeros((1, D), jnp.float32)], 0)
    t1 = jnp.concatenate([e1_1[r5] + e2_1[r3], jnp.zeros((1, D), jnp.float32)], 0)

    h0, cnt_flat = _pre(emb, xi2d, dst2d, ea02d, ea12d)
    cnt3 = cnt_flat.reshape(NC, NPAD, NCOMBO)

    p0 = _edge_agg(h0, src2d, dst2d)
    h2a, stats0 = _mlp(p0, h0, cnt3, t0, W1_0, b1_0.reshape(1, -1), W2_0,
                       b2_0.reshape(1, -1))
    h1 = _bn(True, h2a, stats0, g_0.reshape(1, -1), be_0.reshape(1, -1))

    p1 = _edge_agg(h1, src2d, dst2d)
    h2b, stats1 = _mlp(p1, h1, cnt3, t1, W1_1, b1_1.reshape(1, -1), W2_1,
                       b2_1.reshape(1, -1))
    out = _bn(False, h2b, stats1, g_1.reshape(1, -1), be_1.reshape(1, -1))
    return out[:NNODE]
